# Initial kernel scaffold; baseline (speedup 1.0000x reference)
import jax
import jax.numpy as jnp
from jax.experimental import pallas as pl


def kernel(x_img, text, Wg, W1, b1, W2, b2):
    raise NotImplementedError("write your pallas kernel here")


# speedup vs baseline: 1.7954x; 1.7954x over previous
import functools

import jax
import jax.numpy as jnp
from jax import lax
from jax.experimental import pallas as pl
from jax.experimental.pallas import tpu as pltpu
from jax.experimental.pallas import tpu_sc as plsc

E = 16
S = 2048
D = 768
BLK = 128
NB = S // BLK
NP = 32
NW = 32
CH = S // NW

_HIGHEST = lax.Precision.HIGHEST


def _row2col(x_row, n):
    eye = (lax.broadcasted_iota(jnp.int32, (n, n), 0) ==
           lax.broadcasted_iota(jnp.int32, (n, n), 1))
    return jnp.sum(jnp.where(eye, x_row, 0.0), axis=1, keepdims=True)


def _gate_meta_body(x_ref, wg_ref, pos_ref, meta_ref, loss_ref):
    x = x_ref[...]
    wg = wg_ref[...]
    logits = lax.dot_general(x, wg, (((1,), (1,)), ((), ())),
                             preferred_element_type=jnp.float32)
    m = jnp.max(logits, axis=-1, keepdims=True)
    ex = jnp.exp(logits - m)
    se = jnp.sum(ex, axis=-1, keepdims=True)
    sm = ex / se
    z = m + jnp.log(se)
    rzl = jnp.mean(jnp.square(z))

    vmax = jnp.max(sm, axis=-1, keepdims=True)
    eio = lax.broadcasted_iota(jnp.int32, (S, E), 1)
    sel = jnp.min(jnp.where(sm == vmax, eio, E), axis=-1, keepdims=True)
    onehot = (eio == sel).astype(jnp.int32)

    cum = onehot
    k = 1
    while k < S:
        cum = cum + jnp.concatenate(
            [jnp.zeros((k, E), jnp.int32), cum[:S - k]], axis=0)
        k *= 2
    counts = cum[S - 1:S, :]
    rank = jnp.sum(jnp.where(onehot == 1, cum, 0), axis=-1, keepdims=True) - 1

    proxy = jnp.mean(sm, axis=0, keepdims=True)
    density = counts.astype(jnp.float32) / float(S)
    balance = jnp.sum(proxy * density) * float(E)

    tri_e = (lax.broadcasted_iota(jnp.int32, (E, E), 0) <
             lax.broadcasted_iota(jnp.int32, (E, E), 1)).astype(jnp.float32)
    cf = counts.astype(jnp.float32)
    offs = lax.dot_general(cf, tri_e, (((1,), (0,)), ((), ())),
                           preferred_element_type=jnp.float32,
                           precision=_HIGHEST)
    ends = offs + cf

    offs_tok = jnp.sum(jnp.where(onehot == 1, offs, 0.0), axis=-1,
                       keepdims=True)
    pos_ref[...] = offs_tok.astype(jnp.int32) + rank

    bio = lax.broadcasted_iota(jnp.int32, (NB, E), 0).astype(jnp.float32)
    lo = bio * float(BLK)
    hi = lo + float(BLK)
    mbe = jnp.logical_and(offs < hi, ends > lo)
    mf = mbe.astype(jnp.float32)
    rc = lax.dot_general(mf, tri_e, (((1,), (0,)), ((), ())),
                         preferred_element_type=jnp.float32,
                         precision=_HIGHEST)
    rowsum = jnp.sum(mf, axis=-1, keepdims=True)
    tri_b = (lax.broadcasted_iota(jnp.int32, (NB, NB), 0) <
             lax.broadcasted_iota(jnp.int32, (NB, NB), 1)).astype(jnp.float32)
    rowoff = lax.dot_general(tri_b, rowsum, (((0,), (0,)), ((), ())),
                             preferred_element_type=jnp.float32,
                             precision=_HIGHEST)
    ci = rc + rowoff
    npairs = jnp.sum(mf)
    firstf = (rc == 0.0).astype(jnp.float32)

    pio = lax.broadcasted_iota(jnp.int32, (E, NP), 1).astype(jnp.float32)
    evals = lax.broadcasted_iota(jnp.int32, (1, E), 1).astype(jnp.float32)
    mb = jnp.zeros((1, NP), jnp.float32)
    me = jnp.zeros((1, NP), jnp.float32)
    mfirst = jnp.zeros((1, NP), jnp.float32)
    for b in range(NB):
        ci_col = _row2col(ci[b:b + 1, :], E)
        m_col = _row2col(mf[b:b + 1, :], E)
        ind = jnp.where(jnp.logical_and(ci_col == pio, m_col > 0.0), 1.0, 0.0)
        mb = mb + float(b) * jnp.sum(ind, axis=0, keepdims=True)
        me = me + lax.dot_general(evals, ind, (((1,), (0,)), ((), ())),
                                  preferred_element_type=jnp.float32,
                                  precision=_HIGHEST)
        mfirst = mfirst + lax.dot_general(
            firstf[b:b + 1, :], ind, (((1,), (0,)), ((), ())),
            preferred_element_type=jnp.float32, precision=_HIGHEST)

    iota_np = lax.broadcasted_iota(jnp.int32, (1, NP), 1).astype(jnp.float32)
    validv = (iota_np < npairs).astype(jnp.int32)
    lastsel = iota_np == (npairs - 1.0)
    lastb = jnp.sum(jnp.where(lastsel, mb, 0.0))
    laste = jnp.sum(jnp.where(lastsel, me, 0.0))
    mb = jnp.where(validv == 1, mb, lastb)
    me = jnp.where(validv == 1, me, laste)
    mfirst = jnp.where(validv == 1, mfirst, 0.0)

    offs_pad = jnp.concatenate(
        [offs, jnp.full((1, NP - E), float(S), jnp.float32)], axis=1)
    zrow = jnp.zeros((1, NP), jnp.float32)
    meta = jnp.concatenate(
        [mb, me, mfirst, validv.astype(jnp.float32), offs_pad,
         zrow, zrow, zrow], axis=0)
    meta_ref[...] = meta.astype(jnp.int32)

    loss_ref[...] = jnp.concatenate(
        [jnp.reshape(balance, (1, 1)), jnp.reshape(rzl, (1, 1))], axis=1)


def _mlp_body(meta_ref, xs_ref, w1_ref, b1_ref, w2_ref, b2_ref, ys_ref):
    t = pl.program_id(0)
    b = meta_ref[0, t]
    e = meta_ref[1, t]
    first = meta_ref[2, t]
    valid = meta_ref[3, t]
    off0 = meta_ref[4, e]
    off1 = meta_ref[4, e + 1]
    x = xs_ref[...]
    h = lax.dot_general(x, w1_ref[0], (((1,), (1,)), ((), ())),
                        preferred_element_type=jnp.float32,
                        precision=_HIGHEST) + b1_ref[0]
    h = 0.5 * h * (1.0 + lax.erf(h * 0.7071067811865476))
    o = lax.dot_general(h, w2_ref[0], (((1,), (1,)), ((), ())),
                        preferred_element_type=jnp.float32,
                        precision=_HIGHEST) + b2_ref[0]
    r = b * BLK + lax.broadcasted_iota(jnp.int32, (BLK, 1), 0)
    memb = jnp.logical_and(jnp.logical_and(r >= off0, r < off1), valid > 0)
    contrib = jnp.where(memb, o, 0.0)

    @pl.when(first == 1)
    def _():
        ys_ref[...] = contrib

    @pl.when(first == 0)
    def _():
        ys_ref[...] = ys_ref[...] + contrib


@functools.cache
def _sc_kernels():
    mesh = plsc.VectorSubcoreMesh(core_axis_name="c", subcore_axis_name="s")
    common = dict(
        out_type=jax.ShapeDtypeStruct((S, D), jnp.float32),
        mesh=mesh,
        scratch_types=[pltpu.VMEM((CH,), jnp.int32),
                       pltpu.VMEM((CH, D), jnp.float32),
                       pltpu.SemaphoreType.DMA],
    )

    @functools.partial(pl.kernel, **common)
    def scatter_rows(x_hbm, pos_hbm, out_hbm, idx_v, rows_v, sem):
        wid = lax.axis_index("s") * 2 + lax.axis_index("c")
        base = wid * CH
        pltpu.sync_copy(pos_hbm.at[pl.ds(base, CH)], idx_v)
        pltpu.sync_copy(x_hbm.at[pl.ds(base, CH)], rows_v)
        pltpu.async_copy(rows_v, out_hbm.at[idx_v], sem).wait()

    @functools.partial(pl.kernel, **common)
    def gather_rows(ys_hbm, pos_hbm, out_hbm, idx_v, rows_v, sem):
        wid = lax.axis_index("s") * 2 + lax.axis_index("c")
        base = wid * CH
        pltpu.sync_copy(pos_hbm.at[pl.ds(base, CH)], idx_v)
        pltpu.async_copy(ys_hbm.at[idx_v], rows_v, sem).wait()
        pltpu.sync_copy(rows_v, out_hbm.at[pl.ds(base, CH)])

    return scatter_rows, gather_rows


def kernel(x_img, text, Wg, W1, b1, W2, b2):
    del text
    x = x_img.reshape(S, D)

    pos2, meta, losses = pl.pallas_call(
        _gate_meta_body,
        out_shape=[
            jax.ShapeDtypeStruct((S, 1), jnp.int32),
            jax.ShapeDtypeStruct((8, NP), jnp.int32),
            jax.ShapeDtypeStruct((1, 2), jnp.float32),
        ],
    )(x, Wg)
    pos = pos2.reshape(S)

    xs = jnp.zeros((S, D), jnp.float32).at[pos].set(x)

    ys = pl.pallas_call(
        _mlp_body,
        grid_spec=pltpu.PrefetchScalarGridSpec(
            num_scalar_prefetch=1,
            grid=(NP,),
            in_specs=[
                pl.BlockSpec((BLK, D), lambda t, m: (m[0, t], 0)),
                pl.BlockSpec((1, D, D), lambda t, m: (m[1, t], 0, 0)),
                pl.BlockSpec((1, 1, D), lambda t, m: (m[1, t], 0, 0)),
                pl.BlockSpec((1, D, D), lambda t, m: (m[1, t], 0, 0)),
                pl.BlockSpec((1, 1, D), lambda t, m: (m[1, t], 0, 0)),
            ],
            out_specs=pl.BlockSpec((BLK, D), lambda t, m: (m[0, t], 0)),
        ),
        out_shape=jax.ShapeDtypeStruct((S, D), jnp.float32),
        compiler_params=pltpu.CompilerParams(
            dimension_semantics=("arbitrary",)),
    )(meta, xs, W1, b1.reshape(E, 1, D), W2, b2.reshape(E, 1, D))

    results = jnp.take(ys, pos, axis=0).reshape(1, S, D)
    return results, losses[0, 0], losses[0, 1]


# speedup vs baseline: 1.9106x; 1.0641x over previous
import functools

import jax
import jax.numpy as jnp
from jax import lax
from jax.experimental import pallas as pl
from jax.experimental.pallas import tpu as pltpu
from jax.experimental.pallas import tpu_sc as plsc

E = 16
S = 2048
D = 768
BLK = 128
NB = S // BLK
NP = 32
NW = 32
CH = S // NW

_HIGHEST = lax.Precision.HIGHEST


def _row2col(x_row, n):
    eye = (lax.broadcasted_iota(jnp.int32, (n, n), 0) ==
           lax.broadcasted_iota(jnp.int32, (n, n), 1))
    return jnp.sum(jnp.where(eye, x_row, 0.0), axis=1, keepdims=True)


def _gate_meta_body(x_ref, wg_ref, pos_ref, meta_ref, loss_ref):
    x = x_ref[...]
    wg = wg_ref[...]
    logits = lax.dot_general(x, wg, (((1,), (1,)), ((), ())),
                             preferred_element_type=jnp.float32)
    m = jnp.max(logits, axis=-1, keepdims=True)
    ex = jnp.exp(logits - m)
    se = jnp.sum(ex, axis=-1, keepdims=True)
    sm = ex / se
    z = m + jnp.log(se)
    rzl = jnp.mean(jnp.square(z))

    vmax = jnp.max(sm, axis=-1, keepdims=True)
    eio = lax.broadcasted_iota(jnp.int32, (S, E), 1)
    sel = jnp.min(jnp.where(sm == vmax, eio, E), axis=-1, keepdims=True)
    onehot = (eio == sel).astype(jnp.int32)

    cum = onehot
    k = 1
    while k < S:
        cum = cum + jnp.concatenate(
            [jnp.zeros((k, E), jnp.int32), cum[:S - k]], axis=0)
        k *= 2
    counts = cum[S - 1:S, :]
    rank = jnp.sum(jnp.where(onehot == 1, cum, 0), axis=-1, keepdims=True) - 1

    proxy = jnp.mean(sm, axis=0, keepdims=True)
    density = counts.astype(jnp.float32) / float(S)
    balance = jnp.sum(proxy * density) * float(E)

    tri_e = (lax.broadcasted_iota(jnp.int32, (E, E), 0) <
             lax.broadcasted_iota(jnp.int32, (E, E), 1)).astype(jnp.float32)
    cf = counts.astype(jnp.float32)
    offs = lax.dot_general(cf, tri_e, (((1,), (0,)), ((), ())),
                           preferred_element_type=jnp.float32,
                           precision=_HIGHEST)
    ends = offs + cf

    offs_tok = jnp.sum(jnp.where(onehot == 1, offs, 0.0), axis=-1,
                       keepdims=True)
    pos_ref[...] = offs_tok.astype(jnp.int32) + rank

    bio = lax.broadcasted_iota(jnp.int32, (NB, E), 0).astype(jnp.float32)
    lo = bio * float(BLK)
    hi = lo + float(BLK)
    mbe = jnp.logical_and(offs < hi, ends > lo)
    mf = mbe.astype(jnp.float32)
    rc = lax.dot_general(mf, tri_e, (((1,), (0,)), ((), ())),
                         preferred_element_type=jnp.float32,
                         precision=_HIGHEST)
    rowsum = jnp.sum(mf, axis=-1, keepdims=True)
    tri_b = (lax.broadcasted_iota(jnp.int32, (NB, NB), 0) <
             lax.broadcasted_iota(jnp.int32, (NB, NB), 1)).astype(jnp.float32)
    rowoff = lax.dot_general(tri_b, rowsum, (((0,), (0,)), ((), ())),
                             preferred_element_type=jnp.float32,
                             precision=_HIGHEST)
    ci = rc + rowoff
    npairs = jnp.sum(mf)
    firstf = (rc == 0.0).astype(jnp.float32)

    pio = lax.broadcasted_iota(jnp.int32, (E, NP), 1).astype(jnp.float32)
    evals = lax.broadcasted_iota(jnp.int32, (1, E), 1).astype(jnp.float32)
    mb = jnp.zeros((1, NP), jnp.float32)
    me = jnp.zeros((1, NP), jnp.float32)
    mfirst = jnp.zeros((1, NP), jnp.float32)
    for b in range(NB):
        ci_col = _row2col(ci[b:b + 1, :], E)
        m_col = _row2col(mf[b:b + 1, :], E)
        ind = jnp.where(jnp.logical_and(ci_col == pio, m_col > 0.0), 1.0, 0.0)
        mb = mb + float(b) * jnp.sum(ind, axis=0, keepdims=True)
        me = me + lax.dot_general(evals, ind, (((1,), (0,)), ((), ())),
                                  preferred_element_type=jnp.float32,
                                  precision=_HIGHEST)
        mfirst = mfirst + lax.dot_general(
            firstf[b:b + 1, :], ind, (((1,), (0,)), ((), ())),
            preferred_element_type=jnp.float32, precision=_HIGHEST)

    iota_np = lax.broadcasted_iota(jnp.int32, (1, NP), 1).astype(jnp.float32)
    validv = (iota_np < npairs).astype(jnp.int32)
    lastsel = iota_np == (npairs - 1.0)
    lastb = jnp.sum(jnp.where(lastsel, mb, 0.0))
    laste = jnp.sum(jnp.where(lastsel, me, 0.0))
    mb = jnp.where(validv == 1, mb, lastb)
    me = jnp.where(validv == 1, me, laste)
    mfirst = jnp.where(validv == 1, mfirst, 0.0)

    offs_pad = jnp.concatenate(
        [offs, jnp.full((1, NP - E), float(S), jnp.float32)], axis=1)
    zrow = jnp.zeros((1, NP), jnp.float32)
    meta = jnp.concatenate(
        [mb, me, mfirst, validv.astype(jnp.float32), offs_pad,
         zrow, zrow, zrow], axis=0)
    meta_ref[...] = meta.astype(jnp.int32)

    loss_ref[...] = jnp.concatenate(
        [jnp.reshape(balance, (1, 1)), jnp.reshape(rzl, (1, 1))], axis=1)


def _mlp_body(meta_ref, xs_ref, w1_ref, b1_ref, w2_ref, b2_ref, ys_ref):
    t = pl.program_id(0)
    b = meta_ref[0, t]
    e = meta_ref[1, t]
    first = meta_ref[2, t]
    valid = meta_ref[3, t]
    off0 = meta_ref[4, e]
    off1 = meta_ref[4, e + 1]
    x = xs_ref[...]
    h = lax.dot_general(x, w1_ref[0], (((1,), (1,)), ((), ())),
                        preferred_element_type=jnp.float32,
                        precision=_HIGHEST) + b1_ref[0]
    h = 0.5 * h * (1.0 + lax.erf(h * 0.7071067811865476))
    o = lax.dot_general(h, w2_ref[0], (((1,), (1,)), ((), ())),
                        preferred_element_type=jnp.float32,
                        precision=_HIGHEST) + b2_ref[0]
    r = b * BLK + lax.broadcasted_iota(jnp.int32, (BLK, 1), 0)
    memb = jnp.logical_and(jnp.logical_and(r >= off0, r < off1), valid > 0)
    contrib = jnp.where(memb, o, 0.0)

    @pl.when(first == 1)
    def _():
        ys_ref[...] = contrib

    @pl.when(first == 0)
    def _():
        ys_ref[...] = ys_ref[...] + contrib


@functools.cache
def _sc_kernels():
    mesh = plsc.VectorSubcoreMesh(core_axis_name="c", subcore_axis_name="s")
    common = dict(
        out_type=jax.ShapeDtypeStruct((S, D), jnp.float32),
        mesh=mesh,
        scratch_types=[pltpu.VMEM((CH,), jnp.int32),
                       pltpu.VMEM((CH, D), jnp.float32),
                       pltpu.SemaphoreType.DMA],
    )

    @functools.partial(pl.kernel, **common)
    def scatter_rows(x_hbm, pos_hbm, out_hbm, idx_v, rows_v, sem):
        wid = lax.axis_index("s") * 2 + lax.axis_index("c")
        base = wid * CH
        pltpu.sync_copy(pos_hbm.at[pl.ds(base, CH)], idx_v)
        pltpu.sync_copy(x_hbm.at[pl.ds(base, CH)], rows_v)
        pltpu.async_copy(rows_v, out_hbm.at[idx_v], sem).wait()

    @functools.partial(pl.kernel, **common)
    def gather_rows(ys_hbm, pos_hbm, out_hbm, idx_v, rows_v, sem):
        wid = lax.axis_index("s") * 2 + lax.axis_index("c")
        base = wid * CH
        pltpu.sync_copy(pos_hbm.at[pl.ds(base, CH)], idx_v)
        pltpu.async_copy(ys_hbm.at[idx_v], rows_v, sem).wait()
        pltpu.sync_copy(rows_v, out_hbm.at[pl.ds(base, CH)])

    return scatter_rows, gather_rows


def kernel(x_img, text, Wg, W1, b1, W2, b2):
    del text
    x = x_img.reshape(S, D)

    pos2, meta, losses = pl.pallas_call(
        _gate_meta_body,
        out_shape=[
            jax.ShapeDtypeStruct((S, 1), jnp.int32),
            jax.ShapeDtypeStruct((8, NP), jnp.int32),
            jax.ShapeDtypeStruct((1, 2), jnp.float32),
        ],
    )(x, Wg)
    pos = pos2.reshape(S)

    scatter_rows, gather_rows = _sc_kernels()
    xs = scatter_rows(x, pos)

    ys = pl.pallas_call(
        _mlp_body,
        grid_spec=pltpu.PrefetchScalarGridSpec(
            num_scalar_prefetch=1,
            grid=(NP,),
            in_specs=[
                pl.BlockSpec((BLK, D), lambda t, m: (m[0, t], 0)),
                pl.BlockSpec((1, D, D), lambda t, m: (m[1, t], 0, 0)),
                pl.BlockSpec((1, 1, D), lambda t, m: (m[1, t], 0, 0)),
                pl.BlockSpec((1, D, D), lambda t, m: (m[1, t], 0, 0)),
                pl.BlockSpec((1, 1, D), lambda t, m: (m[1, t], 0, 0)),
            ],
            out_specs=pl.BlockSpec((BLK, D), lambda t, m: (m[0, t], 0)),
        ),
        out_shape=jax.ShapeDtypeStruct((S, D), jnp.float32),
        compiler_params=pltpu.CompilerParams(
            dimension_semantics=("arbitrary",)),
    )(meta, xs, W1, b1.reshape(E, 1, D), W2, b2.reshape(E, 1, D))

    results = gather_rows(ys, pos).reshape(1, S, D)
    return results, losses[0, 0], losses[0, 1]


# speedup vs baseline: 4.1323x; 2.1629x over previous
import functools

import jax
import jax.numpy as jnp
from jax import lax
from jax.experimental import pallas as pl
from jax.experimental.pallas import tpu as pltpu
from jax.experimental.pallas import tpu_sc as plsc

E = 16
S = 2048
D = 768
BLK = 128
NB = S // BLK
NP = 32
NW = 32
CH = S // NW

_HIGHEST = lax.Precision.HIGHEST


def _row2col(x_row, n):
    eye = (lax.broadcasted_iota(jnp.int32, (n, n), 0) ==
           lax.broadcasted_iota(jnp.int32, (n, n), 1))
    return jnp.sum(jnp.where(eye, x_row, 0.0), axis=1, keepdims=True)


def _gate_meta_body(x_ref, wg_ref, pos_ref, meta_ref, loss_ref):
    x = x_ref[...]
    wg = wg_ref[...]
    logits = lax.dot_general(x, wg, (((1,), (1,)), ((), ())),
                             preferred_element_type=jnp.float32)
    m = jnp.max(logits, axis=-1, keepdims=True)
    ex = jnp.exp(logits - m)
    se = jnp.sum(ex, axis=-1, keepdims=True)
    sm = ex / se
    z = m + jnp.log(se)
    rzl = jnp.mean(jnp.square(z))

    vmax = jnp.max(sm, axis=-1, keepdims=True)
    eio = lax.broadcasted_iota(jnp.int32, (S, E), 1)
    sel = jnp.min(jnp.where(sm == vmax, eio, E), axis=-1, keepdims=True)
    onehot = (eio == sel).astype(jnp.int32)

    cum = onehot
    k = 1
    while k < S:
        cum = cum + jnp.concatenate(
            [jnp.zeros((k, E), jnp.int32), cum[:S - k]], axis=0)
        k *= 2
    counts = cum[S - 1:S, :]
    rank = jnp.sum(jnp.where(onehot == 1, cum, 0), axis=-1, keepdims=True) - 1

    proxy = jnp.mean(sm, axis=0, keepdims=True)
    density = counts.astype(jnp.float32) / float(S)
    balance = jnp.sum(proxy * density) * float(E)

    tri_e = (lax.broadcasted_iota(jnp.int32, (E, E), 0) <
             lax.broadcasted_iota(jnp.int32, (E, E), 1)).astype(jnp.float32)
    cf = counts.astype(jnp.float32)
    offs = lax.dot_general(cf, tri_e, (((1,), (0,)), ((), ())),
                           preferred_element_type=jnp.float32,
                           precision=_HIGHEST)
    ends = offs + cf

    offs_tok = jnp.sum(jnp.where(onehot == 1, offs, 0.0), axis=-1,
                       keepdims=True)
    pos_ref[...] = offs_tok.astype(jnp.int32) + rank

    bio = lax.broadcasted_iota(jnp.int32, (NB, E), 0).astype(jnp.float32)
    lo = bio * float(BLK)
    hi = lo + float(BLK)
    mbe = jnp.logical_and(offs < hi, ends > lo)
    mf = mbe.astype(jnp.float32)
    rc = lax.dot_general(mf, tri_e, (((1,), (0,)), ((), ())),
                         preferred_element_type=jnp.float32,
                         precision=_HIGHEST)
    rowsum = jnp.sum(mf, axis=-1, keepdims=True)
    tri_b = (lax.broadcasted_iota(jnp.int32, (NB, NB), 0) <
             lax.broadcasted_iota(jnp.int32, (NB, NB), 1)).astype(jnp.float32)
    rowoff = lax.dot_general(tri_b, rowsum, (((0,), (0,)), ((), ())),
                             preferred_element_type=jnp.float32,
                             precision=_HIGHEST)
    ci = rc + rowoff
    npairs = jnp.sum(mf)
    firstf = (rc == 0.0).astype(jnp.float32)

    pio = lax.broadcasted_iota(jnp.int32, (E, NP), 1).astype(jnp.float32)
    evals = lax.broadcasted_iota(jnp.int32, (1, E), 1).astype(jnp.float32)
    mb = jnp.zeros((1, NP), jnp.float32)
    me = jnp.zeros((1, NP), jnp.float32)
    mfirst = jnp.zeros((1, NP), jnp.float32)
    for b in range(NB):
        ci_col = _row2col(ci[b:b + 1, :], E)
        m_col = _row2col(mf[b:b + 1, :], E)
        ind = jnp.where(jnp.logical_and(ci_col == pio, m_col > 0.0), 1.0, 0.0)
        mb = mb + float(b) * jnp.sum(ind, axis=0, keepdims=True)
        me = me + lax.dot_general(evals, ind, (((1,), (0,)), ((), ())),
                                  preferred_element_type=jnp.float32,
                                  precision=_HIGHEST)
        mfirst = mfirst + lax.dot_general(
            firstf[b:b + 1, :], ind, (((1,), (0,)), ((), ())),
            preferred_element_type=jnp.float32, precision=_HIGHEST)

    iota_np = lax.broadcasted_iota(jnp.int32, (1, NP), 1).astype(jnp.float32)
    validv = (iota_np < npairs).astype(jnp.int32)
    lastsel = iota_np == (npairs - 1.0)
    lastb = jnp.sum(jnp.where(lastsel, mb, 0.0))
    laste = jnp.sum(jnp.where(lastsel, me, 0.0))
    mb = jnp.where(validv == 1, mb, lastb)
    me = jnp.where(validv == 1, me, laste)
    mfirst = jnp.where(validv == 1, mfirst, 0.0)

    offs_pad = jnp.concatenate(
        [offs, jnp.full((1, NP - E), float(S), jnp.float32)], axis=1)
    zrow = jnp.zeros((1, NP), jnp.float32)
    meta = jnp.concatenate(
        [mb, me, mfirst, validv.astype(jnp.float32), offs_pad,
         zrow, zrow, zrow], axis=0)
    meta_ref[...] = meta.astype(jnp.int32)

    loss_ref[...] = jnp.concatenate(
        [jnp.reshape(balance, (1, 1)), jnp.reshape(rzl, (1, 1))], axis=1)


def _mlp_body(meta_ref, xs_ref, w1_ref, b1_ref, w2_ref, b2_ref, ys_ref):
    t = pl.program_id(0)
    b = meta_ref[0, t]
    e = meta_ref[1, t]
    first = meta_ref[2, t]
    valid = meta_ref[3, t]
    off0 = meta_ref[4, e]
    off1 = meta_ref[4, e + 1]
    x = xs_ref[...]
    h = lax.dot_general(x, w1_ref[0], (((1,), (1,)), ((), ())),
                        preferred_element_type=jnp.float32) + b1_ref[0]
    h = 0.5 * h * (1.0 + lax.erf(h * 0.7071067811865476))
    o = lax.dot_general(h, w2_ref[0], (((1,), (1,)), ((), ())),
                        preferred_element_type=jnp.float32) + b2_ref[0]
    r = b * BLK + lax.broadcasted_iota(jnp.int32, (BLK, 1), 0)
    memb = jnp.logical_and(jnp.logical_and(r >= off0, r < off1), valid > 0)
    contrib = jnp.where(memb, o, 0.0)

    @pl.when(first == 1)
    def _():
        ys_ref[...] = contrib

    @pl.when(first == 0)
    def _():
        ys_ref[...] = ys_ref[...] + contrib


@functools.cache
def _sc_kernels():
    mesh = plsc.VectorSubcoreMesh(core_axis_name="c", subcore_axis_name="s")
    common = dict(
        out_type=jax.ShapeDtypeStruct((S, D), jnp.float32),
        mesh=mesh,
        scratch_types=[pltpu.VMEM((CH,), jnp.int32),
                       pltpu.VMEM((CH, D), jnp.float32),
                       pltpu.SemaphoreType.DMA],
    )

    @functools.partial(pl.kernel, **common)
    def scatter_rows(x_hbm, pos_hbm, out_hbm, idx_v, rows_v, sem):
        wid = lax.axis_index("s") * 2 + lax.axis_index("c")
        base = wid * CH
        pltpu.sync_copy(pos_hbm.at[pl.ds(base, CH)], idx_v)
        pltpu.sync_copy(x_hbm.at[pl.ds(base, CH)], rows_v)
        pltpu.async_copy(rows_v, out_hbm.at[idx_v], sem).wait()

    @functools.partial(pl.kernel, **common)
    def gather_rows(ys_hbm, pos_hbm, out_hbm, idx_v, rows_v, sem):
        wid = lax.axis_index("s") * 2 + lax.axis_index("c")
        base = wid * CH
        pltpu.sync_copy(pos_hbm.at[pl.ds(base, CH)], idx_v)
        pltpu.async_copy(ys_hbm.at[idx_v], rows_v, sem).wait()
        pltpu.sync_copy(rows_v, out_hbm.at[pl.ds(base, CH)])

    return scatter_rows, gather_rows


def kernel(x_img, text, Wg, W1, b1, W2, b2):
    del text
    x = x_img.reshape(S, D)

    pos2, meta, losses = pl.pallas_call(
        _gate_meta_body,
        out_shape=[
            jax.ShapeDtypeStruct((S, 1), jnp.int32),
            jax.ShapeDtypeStruct((8, NP), jnp.int32),
            jax.ShapeDtypeStruct((1, 2), jnp.float32),
        ],
    )(x, Wg)
    pos = pos2.reshape(S)

    scatter_rows, gather_rows = _sc_kernels()
    xs = scatter_rows(x, pos)

    ys = pl.pallas_call(
        _mlp_body,
        grid_spec=pltpu.PrefetchScalarGridSpec(
            num_scalar_prefetch=1,
            grid=(NP,),
            in_specs=[
                pl.BlockSpec((BLK, D), lambda t, m: (m[0, t], 0)),
                pl.BlockSpec((1, D, D), lambda t, m: (m[1, t], 0, 0)),
                pl.BlockSpec((1, 1, D), lambda t, m: (m[1, t], 0, 0)),
                pl.BlockSpec((1, D, D), lambda t, m: (m[1, t], 0, 0)),
                pl.BlockSpec((1, 1, D), lambda t, m: (m[1, t], 0, 0)),
            ],
            out_specs=pl.BlockSpec((BLK, D), lambda t, m: (m[0, t], 0)),
        ),
        out_shape=jax.ShapeDtypeStruct((S, D), jnp.float32),
        compiler_params=pltpu.CompilerParams(
            dimension_semantics=("arbitrary",)),
    )(meta, xs, W1, b1.reshape(E, 1, D), W2, b2.reshape(E, 1, D))

    results = gather_rows(ys, pos).reshape(1, S, D)
    return results, losses[0, 0], losses[0, 1]


# speedup vs baseline: 4.8805x; 1.1811x over previous
import functools

import jax
import jax.numpy as jnp
from jax import lax
from jax.experimental import pallas as pl
from jax.experimental.pallas import tpu as pltpu
from jax.experimental.pallas import tpu_sc as plsc

E = 16
S = 2048
D = 768
BLK = 256
NB = S // BLK
NP = 24
NW = 32
CH = S // NW

_HIGHEST = lax.Precision.HIGHEST


def _row2col(x_row, n):
    eye = (lax.broadcasted_iota(jnp.int32, (n, n), 0) ==
           lax.broadcasted_iota(jnp.int32, (n, n), 1))
    return jnp.sum(jnp.where(eye, x_row, 0.0), axis=1, keepdims=True)


def _gate_meta_body(x_ref, wg_ref, pos_ref, meta_ref, loss_ref):
    x = x_ref[...]
    wg = wg_ref[...]
    logits = lax.dot_general(x, wg, (((1,), (1,)), ((), ())),
                             preferred_element_type=jnp.float32)
    m = jnp.max(logits, axis=-1, keepdims=True)
    ex = jnp.exp(logits - m)
    se = jnp.sum(ex, axis=-1, keepdims=True)
    sm = ex / se
    z = m + jnp.log(se)
    rzl = jnp.mean(jnp.square(z))

    vmax = jnp.max(sm, axis=-1, keepdims=True)
    eio = lax.broadcasted_iota(jnp.int32, (S, E), 1)
    sel = jnp.min(jnp.where(sm == vmax, eio, E), axis=-1, keepdims=True)
    onehot = (eio == sel).astype(jnp.int32)

    cum = onehot
    k = 1
    while k < S:
        cum = cum + jnp.concatenate(
            [jnp.zeros((k, E), jnp.int32), cum[:S - k]], axis=0)
        k *= 2
    counts = cum[S - 1:S, :]
    rank = jnp.sum(jnp.where(onehot == 1, cum, 0), axis=-1, keepdims=True) - 1

    proxy = jnp.mean(sm, axis=0, keepdims=True)
    density = counts.astype(jnp.float32) / float(S)
    balance = jnp.sum(proxy * density) * float(E)

    tri_e = (lax.broadcasted_iota(jnp.int32, (E, E), 0) <
             lax.broadcasted_iota(jnp.int32, (E, E), 1)).astype(jnp.float32)
    cf = counts.astype(jnp.float32)
    offs = lax.dot_general(cf, tri_e, (((1,), (0,)), ((), ())),
                           preferred_element_type=jnp.float32,
                           precision=_HIGHEST)
    ends = offs + cf

    offs_tok = jnp.sum(jnp.where(onehot == 1, offs, 0.0), axis=-1,
                       keepdims=True)
    pos_ref[...] = offs_tok.astype(jnp.int32) + rank

    bio = lax.broadcasted_iota(jnp.int32, (NB, E), 0).astype(jnp.float32)
    lo = bio * float(BLK)
    hi = lo + float(BLK)
    mbe = jnp.logical_and(offs < hi, ends > lo)
    mf = mbe.astype(jnp.float32)
    rc = lax.dot_general(mf, tri_e, (((1,), (0,)), ((), ())),
                         preferred_element_type=jnp.float32,
                         precision=_HIGHEST)
    rowsum = jnp.sum(mf, axis=-1, keepdims=True)
    tri_b = (lax.broadcasted_iota(jnp.int32, (NB, NB), 0) <
             lax.broadcasted_iota(jnp.int32, (NB, NB), 1)).astype(jnp.float32)
    rowoff = lax.dot_general(tri_b, rowsum, (((0,), (0,)), ((), ())),
                             preferred_element_type=jnp.float32,
                             precision=_HIGHEST)
    ci = rc + rowoff
    npairs = jnp.sum(mf)
    firstf = (rc == 0.0).astype(jnp.float32)

    pio = lax.broadcasted_iota(jnp.int32, (E, NP), 1).astype(jnp.float32)
    evals = lax.broadcasted_iota(jnp.int32, (1, E), 1).astype(jnp.float32)
    mb = jnp.zeros((1, NP), jnp.float32)
    me = jnp.zeros((1, NP), jnp.float32)
    mfirst = jnp.zeros((1, NP), jnp.float32)
    for b in range(NB):
        ci_col = _row2col(ci[b:b + 1, :], E)
        m_col = _row2col(mf[b:b + 1, :], E)
        ind = jnp.where(jnp.logical_and(ci_col == pio, m_col > 0.0), 1.0, 0.0)
        mb = mb + float(b) * jnp.sum(ind, axis=0, keepdims=True)
        me = me + lax.dot_general(evals, ind, (((1,), (0,)), ((), ())),
                                  preferred_element_type=jnp.float32,
                                  precision=_HIGHEST)
        mfirst = mfirst + lax.dot_general(
            firstf[b:b + 1, :], ind, (((1,), (0,)), ((), ())),
            preferred_element_type=jnp.float32, precision=_HIGHEST)

    iota_np = lax.broadcasted_iota(jnp.int32, (1, NP), 1).astype(jnp.float32)
    validv = (iota_np < npairs).astype(jnp.int32)
    lastsel = iota_np == (npairs - 1.0)
    lastb = jnp.sum(jnp.where(lastsel, mb, 0.0))
    laste = jnp.sum(jnp.where(lastsel, me, 0.0))
    mb = jnp.where(validv == 1, mb, lastb)
    me = jnp.where(validv == 1, me, laste)
    mfirst = jnp.where(validv == 1, mfirst, 0.0)

    offs_pad = jnp.concatenate(
        [offs, jnp.full((1, NP - E), float(S), jnp.float32)], axis=1)
    zrow = jnp.zeros((1, NP), jnp.float32)
    meta = jnp.concatenate(
        [mb, me, mfirst, validv.astype(jnp.float32), offs_pad,
         zrow, zrow, zrow], axis=0)
    meta_ref[...] = meta.astype(jnp.int32)

    loss_ref[...] = jnp.concatenate(
        [jnp.reshape(balance, (1, 1)), jnp.reshape(rzl, (1, 1))], axis=1)


def _mlp_body(meta_ref, xs_ref, w1_ref, b1_ref, w2_ref, b2_ref, ys_ref):
    t = pl.program_id(0)
    b = meta_ref[0, t]
    e = meta_ref[1, t]
    first = meta_ref[2, t]
    valid = meta_ref[3, t]
    off0 = meta_ref[4, e]
    off1 = meta_ref[4, e + 1]
    x = xs_ref[...]
    h = lax.dot_general(x, w1_ref[0], (((1,), (1,)), ((), ())),
                        preferred_element_type=jnp.float32) + b1_ref[0]
    h = 0.5 * h * (1.0 + lax.erf(h * 0.7071067811865476))
    o = lax.dot_general(h, w2_ref[0], (((1,), (1,)), ((), ())),
                        preferred_element_type=jnp.float32) + b2_ref[0]
    r = b * BLK + lax.broadcasted_iota(jnp.int32, (BLK, 1), 0)
    memb = jnp.logical_and(jnp.logical_and(r >= off0, r < off1), valid > 0)
    contrib = jnp.where(memb, o, 0.0)

    @pl.when(first == 1)
    def _():
        ys_ref[...] = contrib

    @pl.when(first == 0)
    def _():
        ys_ref[...] = ys_ref[...] + contrib


@functools.cache
def _sc_kernels():
    mesh = plsc.VectorSubcoreMesh(core_axis_name="c", subcore_axis_name="s")
    common = dict(
        out_type=jax.ShapeDtypeStruct((S, D), jnp.float32),
        mesh=mesh,
        scratch_types=[pltpu.VMEM((CH,), jnp.int32),
                       pltpu.VMEM((CH, D), jnp.float32),
                       pltpu.SemaphoreType.DMA],
    )

    @functools.partial(pl.kernel, **common)
    def scatter_rows(x_hbm, pos_hbm, out_hbm, idx_v, rows_v, sem):
        wid = lax.axis_index("s") * 2 + lax.axis_index("c")
        base = wid * CH
        pltpu.sync_copy(pos_hbm.at[pl.ds(base, CH)], idx_v)
        pltpu.sync_copy(x_hbm.at[pl.ds(base, CH)], rows_v)
        pltpu.async_copy(rows_v, out_hbm.at[idx_v], sem).wait()

    @functools.partial(pl.kernel, **common)
    def gather_rows(ys_hbm, pos_hbm, out_hbm, idx_v, rows_v, sem):
        wid = lax.axis_index("s") * 2 + lax.axis_index("c")
        base = wid * CH
        pltpu.sync_copy(pos_hbm.at[pl.ds(base, CH)], idx_v)
        pltpu.async_copy(ys_hbm.at[idx_v], rows_v, sem).wait()
        pltpu.sync_copy(rows_v, out_hbm.at[pl.ds(base, CH)])

    return scatter_rows, gather_rows


def kernel(x_img, text, Wg, W1, b1, W2, b2):
    del text
    x = x_img.reshape(S, D)

    pos2, meta, losses = pl.pallas_call(
        _gate_meta_body,
        out_shape=[
            jax.ShapeDtypeStruct((S, 1), jnp.int32),
            jax.ShapeDtypeStruct((8, NP), jnp.int32),
            jax.ShapeDtypeStruct((1, 2), jnp.float32),
        ],
    )(x, Wg)
    pos = pos2.reshape(S)

    scatter_rows, gather_rows = _sc_kernels()
    xs = scatter_rows(x, pos)

    ys = pl.pallas_call(
        _mlp_body,
        grid_spec=pltpu.PrefetchScalarGridSpec(
            num_scalar_prefetch=1,
            grid=(NP,),
            in_specs=[
                pl.BlockSpec((BLK, D), lambda t, m: (m[0, t], 0)),
                pl.BlockSpec((1, D, D), lambda t, m: (m[1, t], 0, 0)),
                pl.BlockSpec((1, 1, D), lambda t, m: (m[1, t], 0, 0)),
                pl.BlockSpec((1, D, D), lambda t, m: (m[1, t], 0, 0)),
                pl.BlockSpec((1, 1, D), lambda t, m: (m[1, t], 0, 0)),
            ],
            out_specs=pl.BlockSpec((BLK, D), lambda t, m: (m[0, t], 0)),
        ),
        out_shape=jax.ShapeDtypeStruct((S, D), jnp.float32),
        compiler_params=pltpu.CompilerParams(
            dimension_semantics=("arbitrary",)),
    )(meta, xs, W1, b1.reshape(E, 1, D), W2, b2.reshape(E, 1, D))

    results = gather_rows(ys, pos).reshape(1, S, D)
    return results, losses[0, 0], losses[0, 1]
